# baseline (speedup 1.0000x reference)
import jax
import jax.numpy as jnp
from jax.experimental import pallas as pl


def _bcast_kernel(pe_ref, out_ref):
    out_ref[...] = jnp.broadcast_to(pe_ref[...][None, :, :], out_ref.shape)


def kernel(x, pe_weight):
    batch = x.shape[0]
    max_len, d_model = pe_weight.shape
    bb = 128
    return pl.pallas_call(
        _bcast_kernel,
        grid=(batch // bb,),
        in_specs=[pl.BlockSpec((max_len, d_model), lambda i: (0, 0))],
        out_specs=pl.BlockSpec((bb, max_len, d_model), lambda i: (i, 0, 0)),
        out_shape=jax.ShapeDtypeStruct((batch, max_len, d_model), pe_weight.dtype),
    )(pe_weight)


# speedup vs baseline: 1.6319x; 1.6319x over previous
import jax
import jax.numpy as jnp
from jax.experimental import pallas as pl


def _bcast_kernel(pe_ref, out_ref):
    out_ref[...] = jnp.broadcast_to(pe_ref[...], out_ref.shape)


def kernel(x, pe_weight):
    batch = x.shape[0]
    max_len, d_model = pe_weight.shape
    flat = max_len * d_model
    bb = 128
    out2d = pl.pallas_call(
        _bcast_kernel,
        grid=(batch // bb,),
        in_specs=[pl.BlockSpec((1, flat), lambda i: (0, 0))],
        out_specs=pl.BlockSpec((bb, flat), lambda i: (i, 0)),
        out_shape=jax.ShapeDtypeStruct((batch, flat), pe_weight.dtype),
    )(pe_weight.reshape(1, flat))
    return out2d.reshape(batch, max_len, d_model)
